# baseline (speedup 1.0000x reference)
import jax
import jax.numpy as jnp
from jax.experimental import pallas as pl

NUM_CLASSES_ = 1000
PADDED_ = 1024
N_ = 16384
BLOCK_ROWS = 2048


def _onehot_block(x_ref, o_ref):
    xb = x_ref[0, 0, :]
    col = jax.lax.broadcasted_iota(jnp.int32, (BLOCK_ROWS, PADDED_), 1)
    o_ref[:, :] = (xb[:, None] == col).astype(jnp.int8)


def kernel(x):
    nb = N_ // BLOCK_ROWS
    x3 = x.astype(jnp.int32).reshape(nb, 1, BLOCK_ROWS)
    ind = pl.pallas_call(
        _onehot_block,
        grid=(nb,),
        in_specs=[pl.BlockSpec((1, 1, BLOCK_ROWS), lambda i: (i, 0, 0))],
        out_specs=pl.BlockSpec((BLOCK_ROWS, PADDED_), lambda i: (i, 0)),
        out_shape=jax.ShapeDtypeStruct((N_, PADDED_), jnp.int8),
    )(x3)
    return ind[:, :NUM_CLASSES_].astype(jnp.float32)


# speedup vs baseline: 1.1763x; 1.1763x over previous
import jax
import jax.numpy as jnp
from jax.experimental import pallas as pl

NUM_CLASSES_ = 1000
PADDED_ = 1024
N_ = 16384
BLOCK_ROWS = 2048


def _onehot_block(x_ref, o_ref):
    xb = x_ref[0, 0, :]
    col = jax.lax.broadcasted_iota(jnp.int32, (BLOCK_ROWS, PADDED_), 1)
    o_ref[:, :] = (xb[:, None] == col).astype(jnp.float32)


def kernel(x):
    nb = N_ // BLOCK_ROWS
    x3 = x.astype(jnp.int32).reshape(nb, 1, BLOCK_ROWS)
    ind = pl.pallas_call(
        _onehot_block,
        grid=(nb,),
        in_specs=[pl.BlockSpec((1, 1, BLOCK_ROWS), lambda i: (i, 0, 0))],
        out_specs=pl.BlockSpec((BLOCK_ROWS, PADDED_), lambda i: (i, 0)),
        out_shape=jax.ShapeDtypeStruct((N_, PADDED_), jnp.float32),
    )(x3)
    return ind[:, :NUM_CLASSES_].astype(jnp.float32)


# speedup vs baseline: 1.2540x; 1.0660x over previous
import jax
import jax.numpy as jnp
from jax.experimental import pallas as pl
from jax.experimental.pallas import tpu as pltpu

NUM_CLASSES_ = 1000
N_ = 16384
BLOCK_ROWS = 4096


def _onehot_block(x_ref, o_ref):
    xb = x_ref[0, 0, :]
    col = jax.lax.broadcasted_iota(jnp.int32, (BLOCK_ROWS, NUM_CLASSES_), 1)
    o_ref[:, :] = (xb[:, None] == col).astype(jnp.float32)


def kernel(x):
    nb = N_ // BLOCK_ROWS
    x3 = x.astype(jnp.int32).reshape(nb, 1, BLOCK_ROWS)
    out = pl.pallas_call(
        _onehot_block,
        grid=(nb,),
        in_specs=[pl.BlockSpec((1, 1, BLOCK_ROWS), lambda i: (i, 0, 0))],
        out_specs=pl.BlockSpec((BLOCK_ROWS, NUM_CLASSES_), lambda i: (i, 0)),
        out_shape=jax.ShapeDtypeStruct((N_, NUM_CLASSES_), jnp.float32),
    )(x3)
    return out
